# baseline (speedup 1.0000x reference)
import functools

import jax
import jax.numpy as jnp
from jax import lax
from jax.experimental import pallas as pl
from jax.experimental.pallas import tpu as pltpu
from jax.experimental.pallas import tpu_sc as plsc

BS = 4
NUM_ROWS = 256
NUM_XS = 64
LANES = 16
NUM_GROUPS = BS * NUM_ROWS // LANES
NUM_CORES = 2
NUM_SUBCORES = 16
NUM_WORKERS = NUM_CORES * NUM_SUBCORES
GROUPS_PER_WORKER = NUM_GROUPS // NUM_WORKERS
SLOPE = 0.2
CHAINS = 8


def _attn_layer(f_ref, ca, cb, cc, out_ref):
    zero = jnp.zeros((LANES,), jnp.float32)

    def j_body(j, acc):
        bjc = f_ref[j] * cb + cc

        den = [zero] * CHAINS
        num = [zero] * CHAINS
        for i in range(NUM_XS):
            fi = f_ref[i]
            z = fi * ca + bjc
            l = jnp.maximum(z, SLOPE * z)
            e = jnp.exp(l)
            k = i % CHAINS
            den[k] = den[k] + e
            num[k] = num[k] + e * fi
        while len(den) > 1:
            den = [den[k] + den[k + 1] for k in range(0, len(den), 2)]
            num = [num[k] + num[k + 1] for k in range(0, len(num), 2)]
        sj = num[0] / (den[0] + 1e-16)
        if out_ref is None:
            return acc + sj
        out_ref[j] = sj
        return acc

    return lax.fori_loop(0, NUM_XS, j_body, zero)


def _make_sc_forward():
    mesh = plsc.VectorSubcoreMesh(core_axis_name="c", subcore_axis_name="s")

    @functools.partial(
        pl.kernel,
        mesh=mesh,
        out_type=jax.ShapeDtypeStruct((NUM_GROUPS, LANES), jnp.float32),
        scratch_types=[
            pltpu.VMEM((8, LANES), jnp.float32),
            pltpu.VMEM((NUM_XS, LANES), jnp.float32),
            pltpu.VMEM((NUM_XS, LANES), jnp.float32),
            pltpu.VMEM((LANES,), jnp.float32),
        ],
    )
    def sc_forward(xt_hbm, coef_hbm, out_hbm, coef_vm, x_vm, s_vm, t_vm):
        wid = lax.axis_index("s") * NUM_CORES + lax.axis_index("c")
        b = wid // (NUM_WORKERS // BS)
        pltpu.sync_copy(coef_hbm.at[b], coef_vm)
        ca1 = coef_vm[0]
        cb1 = coef_vm[1]
        ca2 = coef_vm[2]
        cb2 = coef_vm[3]
        cc2 = coef_vm[4]
        zero = jnp.zeros((LANES,), jnp.float32)

        for gg in range(GROUPS_PER_WORKER):
            g = wid * GROUPS_PER_WORKER + gg
            pltpu.sync_copy(xt_hbm.at[g], x_vm)
            _attn_layer(x_vm, ca1, cb1, zero, s_vm)
            t_vm[...] = _attn_layer(s_vm, ca2, cb2, cc2, None)
            pltpu.sync_copy(t_vm, out_hbm.at[g])

    return sc_forward


_sc_forward = _make_sc_forward()


def kernel(xs, lin1, src1, dst1, bias1, lin2, src2, dst2, bias2, W_out, b_out):
    bs, num_rows, num_xs = xs.shape

    w1 = lin1[:, :, 0]
    cs1 = jnp.einsum("bi,bi->b", w1, src1)
    cd1 = jnp.einsum("bi,bi->b", w1, dst1)
    u = jnp.einsum("bij,bj->bi", lin2, w1)
    v = jnp.einsum("bij,bj->bi", lin2, bias1)
    p2 = jnp.einsum("bi,bi->b", u, src2)
    pc2 = jnp.einsum("bi,bi->b", v, src2)
    q2 = jnp.einsum("bi,bi->b", u, dst2)
    qc2 = jnp.einsum("bi,bi->b", v, dst2)
    coef = jnp.stack(
        [cs1, cd1, p2, q2, pc2 + qc2,
         jnp.zeros_like(cs1), jnp.zeros_like(cs1), jnp.zeros_like(cs1)],
        axis=1,
    )
    coef_b = jnp.broadcast_to(coef[:, :, None], (bs, 8, LANES))

    xt = xs.reshape(bs, num_rows // LANES, LANES, num_xs)
    xt = xt.transpose(0, 1, 3, 2).reshape(NUM_GROUPS, num_xs, LANES)

    T = _sc_forward(xt, coef_b)
    T = T.reshape(bs, num_rows)

    g2 = jnp.einsum("bi,oi->bo", u, W_out)
    c2 = jnp.einsum("bi,oi->bo", num_xs * (v + bias2), W_out) + b_out
    return T[:, :, None] * g2[:, None, :] + c2[:, None, :]


# speedup vs baseline: 1.4861x; 1.4861x over previous
import functools

import jax
import jax.numpy as jnp
from jax import lax
from jax.experimental import pallas as pl
from jax.experimental.pallas import tpu as pltpu
from jax.experimental.pallas import tpu_sc as plsc

BS = 4
NUM_ROWS = 256
NUM_XS = 64
LANES = 16
NUM_GROUPS = BS * NUM_ROWS // LANES
NUM_CORES = 2
NUM_SUBCORES = 16
NUM_WORKERS = NUM_CORES * NUM_SUBCORES
GROUPS_PER_WORKER = NUM_GROUPS // NUM_WORKERS
SLOPE = 0.2
UNROLL = 8


def _attn_layer(f_ref, a_ref, ca, cb, cc, out_ref):
    zero = jnp.zeros((LANES,), jnp.float32)

    def a_body(i, _):
        a_ref[i] = f_ref[i] * ca
        return 0

    lax.fori_loop(0, NUM_XS, a_body, 0)

    def j_body(j, acc):
        bjc = f_ref[j] * cb + cc

        def i_body(ii, c):
            i = ii * UNROLL
            new = []
            for k in range(UNROLL):
                fi = f_ref[i + k]
                z = a_ref[i + k] + bjc
                l = jnp.maximum(z, SLOPE * z)
                e = jnp.exp(l)
                new.append((c[k][0] + e, c[k][1] + e * fi))
            return tuple(new)

        c = lax.fori_loop(
            0, NUM_XS // UNROLL, i_body, ((zero, zero),) * UNROLL
        )
        den = c[0][0]
        num = c[0][1]
        for k in range(1, UNROLL):
            den = den + c[k][0]
            num = num + c[k][1]
        sj = num / (den + 1e-16)
        if out_ref is None:
            return acc + sj
        out_ref[j] = sj
        return acc

    return lax.fori_loop(0, NUM_XS, j_body, zero)


def _make_sc_forward():
    mesh = plsc.VectorSubcoreMesh(core_axis_name="c", subcore_axis_name="s")

    @functools.partial(
        pl.kernel,
        mesh=mesh,
        out_type=jax.ShapeDtypeStruct((NUM_GROUPS, LANES), jnp.float32),
        scratch_types=[
            pltpu.VMEM((8, LANES), jnp.float32),
            pltpu.VMEM((NUM_XS, LANES), jnp.float32),
            pltpu.VMEM((NUM_XS, LANES), jnp.float32),
            pltpu.VMEM((NUM_XS, LANES), jnp.float32),
            pltpu.VMEM((LANES,), jnp.float32),
        ],
    )
    def sc_forward(xt_hbm, coef_hbm, out_hbm, coef_vm, x_vm, a_vm, s_vm, t_vm):
        wid = lax.axis_index("s") * NUM_CORES + lax.axis_index("c")
        b = wid // (NUM_WORKERS // BS)
        pltpu.sync_copy(coef_hbm.at[b], coef_vm)
        ca1 = coef_vm[0]
        cb1 = coef_vm[1]
        ca2 = coef_vm[2]
        cb2 = coef_vm[3]
        cc2 = coef_vm[4]
        zero = jnp.zeros((LANES,), jnp.float32)

        for gg in range(GROUPS_PER_WORKER):
            g = wid * GROUPS_PER_WORKER + gg
            pltpu.sync_copy(xt_hbm.at[g], x_vm)
            _attn_layer(x_vm, a_vm, ca1, cb1, zero, s_vm)
            t_vm[...] = _attn_layer(s_vm, a_vm, ca2, cb2, cc2, None)
            pltpu.sync_copy(t_vm, out_hbm.at[g])

    return sc_forward


_sc_forward = _make_sc_forward()


def kernel(xs, lin1, src1, dst1, bias1, lin2, src2, dst2, bias2, W_out, b_out):
    bs, num_rows, num_xs = xs.shape

    w1 = lin1[:, :, 0]
    cs1 = jnp.einsum("bi,bi->b", w1, src1)
    cd1 = jnp.einsum("bi,bi->b", w1, dst1)
    u = jnp.einsum("bij,bj->bi", lin2, w1)
    v = jnp.einsum("bij,bj->bi", lin2, bias1)
    p2 = jnp.einsum("bi,bi->b", u, src2)
    pc2 = jnp.einsum("bi,bi->b", v, src2)
    q2 = jnp.einsum("bi,bi->b", u, dst2)
    qc2 = jnp.einsum("bi,bi->b", v, dst2)
    coef = jnp.stack(
        [cs1, cd1, p2, q2, pc2 + qc2,
         jnp.zeros_like(cs1), jnp.zeros_like(cs1), jnp.zeros_like(cs1)],
        axis=1,
    )
    coef_b = jnp.broadcast_to(coef[:, :, None], (bs, 8, LANES))

    xt = xs.reshape(bs, num_rows // LANES, LANES, num_xs)
    xt = xt.transpose(0, 1, 3, 2).reshape(NUM_GROUPS, num_xs, LANES)

    T = _sc_forward(xt, coef_b)
    T = T.reshape(bs, num_rows)

    g2 = jnp.einsum("bi,oi->bo", u, W_out)
    c2 = jnp.einsum("bi,oi->bo", num_xs * (v + bias2), W_out) + b_out
    return T[:, :, None] * g2[:, None, :] + c2[:, None, :]


# speedup vs baseline: 1.5696x; 1.0562x over previous
import functools

import jax
import jax.numpy as jnp
from jax import lax
from jax.experimental import pallas as pl
from jax.experimental.pallas import tpu as pltpu
from jax.experimental.pallas import tpu_sc as plsc

BS = 4
NUM_ROWS = 256
NUM_XS = 64
LANES = 16
NUM_GROUPS = BS * NUM_ROWS // LANES
NUM_CORES = 2
NUM_SUBCORES = 16
NUM_WORKERS = NUM_CORES * NUM_SUBCORES
GROUPS_PER_WORKER = NUM_GROUPS // NUM_WORKERS
SLOPE = 0.2
UNROLL = 16
CHAINS = 2


def _attn_layer(f_ref, a_ref, ca, cb, cc, out_ref):
    zero = jnp.zeros((LANES,), jnp.float32)

    def a_body(i, _):
        a_ref[i] = f_ref[i] * ca
        return 0

    lax.fori_loop(0, NUM_XS, a_body, 0)

    def j_body(j, acc):
        bjc = f_ref[j] * cb + cc

        def i_body(ii, c):
            i = ii * UNROLL
            new = list(c)
            for k in range(UNROLL):
                fi = f_ref[i + k]
                z = a_ref[i + k] + bjc
                l = jnp.maximum(z, SLOPE * z)
                e = jnp.exp(l)
                kk = k % CHAINS
                new[kk] = (new[kk][0] + e, new[kk][1] + e * fi)
            return tuple(new)

        c = lax.fori_loop(
            0, NUM_XS // UNROLL, i_body, ((zero, zero),) * CHAINS
        )
        den = c[0][0]
        num = c[0][1]
        for k in range(1, CHAINS):
            den = den + c[k][0]
            num = num + c[k][1]
        sj = num / (den + 1e-16)
        if out_ref is None:
            return acc + sj
        out_ref[j] = sj
        return acc

    return lax.fori_loop(0, NUM_XS, j_body, zero)


def _make_sc_forward():
    mesh = plsc.VectorSubcoreMesh(core_axis_name="c", subcore_axis_name="s")

    @functools.partial(
        pl.kernel,
        mesh=mesh,
        out_type=jax.ShapeDtypeStruct((NUM_GROUPS, LANES), jnp.float32),
        scratch_types=[
            pltpu.VMEM((8, LANES), jnp.float32),
            pltpu.VMEM((NUM_XS, LANES), jnp.float32),
            pltpu.VMEM((NUM_XS, LANES), jnp.float32),
            pltpu.VMEM((NUM_XS, LANES), jnp.float32),
            pltpu.VMEM((NUM_XS, LANES), jnp.float32),
            pltpu.VMEM((LANES,), jnp.float32),
            pltpu.SemaphoreType.DMA,
            pltpu.SemaphoreType.DMA,
        ],
    )
    def sc_forward(
        xt_hbm, coef_hbm, out_hbm,
        coef_vm, x0_vm, x1_vm, a_vm, s_vm, t_vm, sem0, sem1,
    ):
        wid = lax.axis_index("s") * NUM_CORES + lax.axis_index("c")
        b = wid // (NUM_WORKERS // BS)
        g0 = wid * GROUPS_PER_WORKER
        cp0 = pltpu.async_copy(xt_hbm.at[g0], x0_vm, sem0)
        cp1 = pltpu.async_copy(xt_hbm.at[g0 + 1], x1_vm, sem1)
        pltpu.sync_copy(coef_hbm.at[b], coef_vm)
        ca1 = coef_vm[0]
        cb1 = coef_vm[1]
        ca2 = coef_vm[2]
        cb2 = coef_vm[3]
        cc2 = coef_vm[4]
        zero = jnp.zeros((LANES,), jnp.float32)

        for gg, (x_vm, cp) in enumerate(((x0_vm, cp0), (x1_vm, cp1))):
            cp.wait()
            _attn_layer(x_vm, a_vm, ca1, cb1, zero, s_vm)
            t_vm[...] = _attn_layer(s_vm, a_vm, ca2, cb2, cc2, None)
            pltpu.sync_copy(t_vm, out_hbm.at[g0 + gg])

    return sc_forward


_sc_forward = _make_sc_forward()


def kernel(xs, lin1, src1, dst1, bias1, lin2, src2, dst2, bias2, W_out, b_out):
    bs, num_rows, num_xs = xs.shape

    w1 = lin1[:, :, 0]
    cs1 = jnp.einsum("bi,bi->b", w1, src1)
    cd1 = jnp.einsum("bi,bi->b", w1, dst1)
    u = jnp.einsum("bij,bj->bi", lin2, w1)
    v = jnp.einsum("bij,bj->bi", lin2, bias1)
    p2 = jnp.einsum("bi,bi->b", u, src2)
    pc2 = jnp.einsum("bi,bi->b", v, src2)
    q2 = jnp.einsum("bi,bi->b", u, dst2)
    qc2 = jnp.einsum("bi,bi->b", v, dst2)
    coef = jnp.stack(
        [cs1, cd1, p2, q2, pc2 + qc2,
         jnp.zeros_like(cs1), jnp.zeros_like(cs1), jnp.zeros_like(cs1)],
        axis=1,
    )
    coef_b = jnp.broadcast_to(coef[:, :, None], (bs, 8, LANES))

    xt = xs.reshape(bs, num_rows // LANES, LANES, num_xs)
    xt = xt.transpose(0, 1, 3, 2).reshape(NUM_GROUPS, num_xs, LANES)

    T = _sc_forward(xt, coef_b)
    T = T.reshape(bs, num_rows)

    g2 = jnp.einsum("bi,oi->bo", u, W_out)
    c2 = jnp.einsum("bi,oi->bo", num_xs * (v + bias2), W_out) + b_out
    return T[:, :, None] * g2[:, None, :] + c2[:, None, :]


# speedup vs baseline: 1.9264x; 1.2273x over previous
import functools

import jax
import jax.numpy as jnp
from jax import lax
from jax.experimental import pallas as pl
from jax.experimental.pallas import tpu as pltpu
from jax.experimental.pallas import tpu_sc as plsc

BS = 4
NUM_ROWS = 256
NUM_XS = 64
LANES = 16
NUM_GROUPS = BS * NUM_ROWS // LANES
NUM_CORES = 2
NUM_SUBCORES = 16
NUM_WORKERS = NUM_CORES * NUM_SUBCORES
SC_BATCHES = 2
SC_GROUPS = SC_BATCHES * NUM_ROWS // LANES
TC_ROW_BLOCK = 64
SLOPE = 0.2
UNROLL = 16
CHAINS = 2


def _attn_layer(f_ref, a_ref, ca, cb, cc, out_ref):
    zero = jnp.zeros((LANES,), jnp.float32)

    def a_body(i, _):
        a_ref[i] = f_ref[i] * ca
        return 0

    lax.fori_loop(0, NUM_XS, a_body, 0)

    def j_body(j, acc):
        bjc = f_ref[j] * cb + cc

        def i_body(ii, c):
            i = ii * UNROLL
            new = list(c)
            for k in range(UNROLL):
                fi = f_ref[i + k]
                z = a_ref[i + k] + bjc
                l = jnp.maximum(z, SLOPE * z)
                e = jnp.exp(l)
                kk = k % CHAINS
                new[kk] = (new[kk][0] + e, new[kk][1] + e * fi)
            return tuple(new)

        c = lax.fori_loop(
            0, NUM_XS // UNROLL, i_body, ((zero, zero),) * CHAINS
        )
        den = c[0][0]
        num = c[0][1]
        for k in range(1, CHAINS):
            den = den + c[k][0]
            num = num + c[k][1]
        sj = num / (den + 1e-16)
        if out_ref is None:
            return acc + sj
        out_ref[j] = sj
        return acc

    return lax.fori_loop(0, NUM_XS, j_body, zero)


def _make_sc_forward():
    mesh = plsc.VectorSubcoreMesh(core_axis_name="c", subcore_axis_name="s")

    @functools.partial(
        pl.kernel,
        mesh=mesh,
        out_type=jax.ShapeDtypeStruct((SC_GROUPS, LANES), jnp.float32),
        scratch_types=[
            pltpu.VMEM((8, LANES), jnp.float32),
            pltpu.VMEM((NUM_XS, LANES), jnp.float32),
            pltpu.VMEM((NUM_XS, LANES), jnp.float32),
            pltpu.VMEM((NUM_XS, LANES), jnp.float32),
            pltpu.VMEM((LANES,), jnp.float32),
            pltpu.SemaphoreType.DMA,
        ],
    )
    def sc_forward(
        xt_hbm, coef_hbm, out_hbm,
        coef_vm, x_vm, a_vm, s_vm, t_vm, sem,
    ):
        wid = lax.axis_index("s") * NUM_CORES + lax.axis_index("c")
        b = wid // (SC_GROUPS // SC_BATCHES)
        cp = pltpu.async_copy(xt_hbm.at[wid], x_vm, sem)
        pltpu.sync_copy(coef_hbm.at[b], coef_vm)
        ca1 = coef_vm[0]
        cb1 = coef_vm[1]
        ca2 = coef_vm[2]
        cb2 = coef_vm[3]
        cc2 = coef_vm[4]
        zero = jnp.zeros((LANES,), jnp.float32)

        cp.wait()
        _attn_layer(x_vm, a_vm, ca1, cb1, zero, s_vm)
        t_vm[...] = _attn_layer(s_vm, a_vm, ca2, cb2, cc2, None)
        pltpu.sync_copy(t_vm, out_hbm.at[wid])

    return sc_forward


_sc_forward = _make_sc_forward()


def _tc_body(coef_ref, x_ref, out_ref):
    coefs = coef_ref[0]
    x = x_ref[0]

    def layer(f, ca, cb, cc):
        a = f * ca
        b_ = f * cb + cc
        z = a[:, :, None] + b_[:, None, :]
        l = jnp.maximum(z, SLOPE * z)
        e = jnp.exp(l)
        den = e.sum(axis=1)
        num = (e * f[:, :, None]).sum(axis=1)
        return num / (den + 1e-16)

    s = layer(x, coefs[0:1, :], coefs[1:2, :], jnp.float32(0.0))
    t = layer(s, coefs[2:3, :], coefs[3:4, :], coefs[4:5, :])
    out_ref[0, 0] = t.sum(axis=1)


def _tc_forward(x_tc, coef_tc):
    b_tc = x_tc.shape[0]
    n_blk = NUM_ROWS // TC_ROW_BLOCK
    out = pl.pallas_call(
        _tc_body,
        grid=(b_tc, n_blk),
        in_specs=[
            pl.BlockSpec((1, 8, NUM_XS), lambda b, r: (b, 0, 0)),
            pl.BlockSpec((1, TC_ROW_BLOCK, NUM_XS), lambda b, r: (b, r, 0)),
        ],
        out_specs=pl.BlockSpec(
            (1, 1, TC_ROW_BLOCK), lambda b, r: (b * n_blk + r, 0, 0)
        ),
        out_shape=jax.ShapeDtypeStruct(
            (b_tc * n_blk, 1, TC_ROW_BLOCK), jnp.float32
        ),
    )(coef_tc, x_tc)
    return out.reshape(b_tc, NUM_ROWS)


def kernel(xs, lin1, src1, dst1, bias1, lin2, src2, dst2, bias2, W_out, b_out):
    bs, num_rows, num_xs = xs.shape

    w1 = lin1[:, :, 0]
    cs1 = jnp.einsum("bi,bi->b", w1, src1)
    cd1 = jnp.einsum("bi,bi->b", w1, dst1)
    u = jnp.einsum("bij,bj->bi", lin2, w1)
    v = jnp.einsum("bij,bj->bi", lin2, bias1)
    p2 = jnp.einsum("bi,bi->b", u, src2)
    pc2 = jnp.einsum("bi,bi->b", v, src2)
    q2 = jnp.einsum("bi,bi->b", u, dst2)
    qc2 = jnp.einsum("bi,bi->b", v, dst2)
    coef = jnp.stack(
        [cs1, cd1, p2, q2, pc2 + qc2,
         jnp.zeros_like(cs1), jnp.zeros_like(cs1), jnp.zeros_like(cs1)],
        axis=1,
    )

    xs_sc = xs[:SC_BATCHES]
    xt = xs_sc.reshape(SC_BATCHES, NUM_ROWS // LANES, LANES, num_xs)
    xt = xt.transpose(0, 1, 3, 2).reshape(SC_GROUPS, num_xs, LANES)
    coef_sc = jnp.broadcast_to(
        coef[:SC_BATCHES, :, None], (SC_BATCHES, 8, LANES)
    )
    T_sc = _sc_forward(xt, coef_sc)
    T_sc = T_sc.reshape(SC_BATCHES, NUM_ROWS)

    coef_tc = jnp.broadcast_to(
        coef[SC_BATCHES:, :, None], (bs - SC_BATCHES, 8, NUM_XS)
    )
    T_tc = _tc_forward(xs[SC_BATCHES:], coef_tc)

    T = jnp.concatenate([T_sc, T_tc], axis=0)

    g2 = jnp.einsum("bi,oi->bo", u, W_out)
    c2 = jnp.einsum("bi,oi->bo", num_xs * (v + bias2), W_out) + b_out
    return T[:, :, None] * g2[:, None, :] + c2[:, None, :]
